# baseline (speedup 1.0000x reference)
import functools

import jax
import jax.numpy as jnp
from jax import lax
from jax.experimental import pallas as pl
from jax.experimental.pallas import tpu as pltpu
from jax.experimental.pallas import tpu_sc as plsc

NC = 2
NS = 16
LANES = 16

NSEL = 1000
F_PAD = 64
NCOL = F_PAD // LANES
HID = 128

N2 = 102400
ROWS_PT = N2 // NS
ZR = 640
EBLK = 2048
SB = 128
T2 = 1024
T2_PT = T2 // NS


def _sc_conv1(src_h, dst2d_h, xc0_h, xc1_h, xc2_h, xc3_h, zeros_h,
              agg_out, src1d, dst2d, rows_buf, sh_agg, *, eblocks):
    c = lax.axis_index("c")
    s = lax.axis_index("s")
    ept = eblocks * EBLK
    tbase = (c * NS + s) * ept
    xc = [xc0_h, xc1_h, xc2_h, xc3_h]

    for p in range(NCOL):
        def _z(z, _):
            pltpu.sync_copy(zeros_h,
                            sh_agg.at[pl.ds(s * ROWS_PT + z * ZR, ZR), :])
            return 0
        lax.fori_loop(0, ROWS_PT // ZR, _z, 0)
        plsc.subcore_barrier()

        def _blk(b, _):
            eb = tbase + b * EBLK
            rb = pl.multiple_of(eb // SB, LANES)
            pltpu.sync_copy(src_h.at[pl.ds(eb, EBLK)], src1d)
            pltpu.sync_copy(dst2d_h.at[pl.ds(rb, LANES), :], dst2d)
            for j in range(EBLK // SB):
                pltpu.sync_copy(xc[p].at[src1d.at[pl.ds(j * SB, SB)]],
                                rows_buf)
                pltpu.sync_copy(rows_buf, sh_agg.at[dst2d.at[j]], add=True)
            return 0
        lax.fori_loop(0, eblocks, _blk, 0)
        plsc.subcore_barrier()

        ob = pl.multiple_of((c * NCOL + p) * N2 + s * ROWS_PT, 8)
        pltpu.sync_copy(sh_agg.at[pl.ds(s * ROWS_PT, ROWS_PT), :],
                        agg_out.at[pl.ds(ob, ROWS_PT), :])
        plsc.subcore_barrier()


def _sc_conv2(src_h, dst_h, h1_h, zeros2d_h,
              agg_out,
              src1d, dst1d, cg_stage, cs_stage, rows_buf,
              sh_agg, *, eblocks2):
    c = lax.axis_index("c")
    s = lax.axis_index("s")
    ept = eblocks2 * EBLK
    tbase = (c * NS + s) * ept

    pltpu.sync_copy(zeros2d_h, sh_agg.at[pl.ds(s * T2_PT, T2_PT), :])
    plsc.subcore_barrier()

    def _blk(b, _):
        eb = tbase + b * EBLK
        pltpu.sync_copy(src_h.at[pl.ds(eb, EBLK)], src1d)
        pltpu.sync_copy(dst_h.at[pl.ds(eb, EBLK)], dst1d)

        def _sub(bb, _):
            for i in range(SB // LANES):
                d = dst1d[pl.ds(bb * SB + i * LANES, LANES)]
                sv = src1d[pl.ds(bb * SB + i * LANES, LANES)]
                m = d < NSEL
                cg_stage[pl.ds(i * LANES, LANES)] = jnp.where(m, sv, 0)
                cs_stage[pl.ds(i * LANES, LANES)] = jnp.where(m, d, NSEL)
            pltpu.sync_copy(h1_h.at[cg_stage], rows_buf)
            pltpu.sync_copy(rows_buf, sh_agg.at[cs_stage], add=True)
            return 0
        lax.fori_loop(0, EBLK // SB, _sub, 0)
        return 0
    lax.fori_loop(0, eblocks2, _blk, 0)
    plsc.subcore_barrier()

    ob = c * T2 + s * T2_PT
    pltpu.sync_copy(sh_agg.at[pl.ds(s * T2_PT, T2_PT), :],
                    agg_out.at[pl.ds(ob, T2_PT), :])


def _tc_h1_body(a0_ref, a1_ref, a2_ref, a3_ref, b0_ref, b1_ref, b2_ref,
                b3_ref, x_ref, wl_ref, wr_ref, bias_ref, out_ref):
    a = jnp.concatenate(
        [a0_ref[...] + b0_ref[...], a1_ref[...] + b1_ref[...],
         a2_ref[...] + b2_ref[...], a3_ref[...] + b3_ref[...]], axis=1)
    cnt = jnp.maximum(a[:, F_PAD - 1:F_PAD], 1.0)
    am = a / cnt
    h = (jnp.dot(am, wl_ref[...], preferred_element_type=jnp.float32)
         + jnp.dot(x_ref[...], wr_ref[...], preferred_element_type=jnp.float32)
         + bias_ref[...])
    out_ref[...] = jnp.maximum(h, 0.0)


def _tc_head_body(agg_ref, cnt_ref, h1t_ref, rmask_ref,
                  wl2_ref, bl2_ref, wr2_ref, w1_ref, b1_ref,
                  w2_ref, b2_ref, w3_ref, b3_ref, out_ref):
    a = agg_ref[0:NSEL, :] + agg_ref[T2:T2 + NSEL, :]
    cnt = jnp.maximum(cnt_ref[...], 1.0)
    am = a / cnt
    h2 = (jnp.dot(am, wl2_ref[...], preferred_element_type=jnp.float32)
          + bl2_ref[...]
          + jnp.dot(h1t_ref[...], wr2_ref[...], preferred_element_type=jnp.float32))
    h2 = h2 * rmask_ref[...]
    y = jnp.maximum(jnp.dot(h2, w1_ref[...], preferred_element_type=jnp.float32)
                    + b1_ref[...], 0.0)
    y = jnp.maximum(jnp.dot(y, w2_ref[...], preferred_element_type=jnp.float32)
                    + b2_ref[...], 0.0)
    out_ref[...] = (jnp.dot(y, w3_ref[...], preferred_element_type=jnp.float32)
                    + b3_ref[...])


def kernel(edge_feat, edge_index, action_range, Wl1, bl1, Wr1, Wl2, bl2, Wr2,
           W1, b1, W2, b2, W3, b3):
    n, feat = edge_feat.shape
    e = edge_index.shape[1]
    f32 = jnp.float32

    grp = NC * NS * EBLK
    epad = ((e + grp - 1) // grp) * grp
    src = jnp.concatenate([edge_index[0], jnp.zeros((epad - e,), jnp.int32)])
    dst = jnp.concatenate([edge_index[1],
                           jnp.full((epad - e,), n, jnp.int32)])
    dst2d = dst.reshape(epad // SB, SB)
    eblocks = epad // (NC * NS * EBLK)

    xpad = jnp.zeros((N2, F_PAD), f32).at[:n, :feat].set(edge_feat)
    xpad = xpad.at[:, F_PAD - 1].set(1.0)
    xcols = [xpad[:, p * LANES:(p + 1) * LANES] for p in range(NCOL)]

    mesh = plsc.VectorSubcoreMesh(core_axis_name="c", subcore_axis_name="s",
                                  num_cores=NC, num_subcores=NS)

    conv1 = pl.kernel(
        functools.partial(_sc_conv1, eblocks=eblocks),
        out_type=jax.ShapeDtypeStruct((2 * NCOL * N2, LANES), f32),
        mesh=mesh,
        compiler_params=pltpu.CompilerParams(use_tc_tiling_on_sc=False),
        scratch_types=[
            pltpu.VMEM((EBLK,), jnp.int32),
            pltpu.VMEM((LANES, SB), jnp.int32),
            pltpu.VMEM((SB, LANES), f32),
            pltpu.VMEM_SHARED((N2, LANES), f32),
        ],
    )
    zeros1 = jnp.zeros((ZR, LANES), f32)
    agg1 = conv1(src, dst2d, xcols[0], xcols[1], xcols[2], xcols[3], zeros1)
    agg1 = agg1.reshape(2, NCOL, N2, LANES)

    wl1t = jnp.zeros((F_PAD, HID), f32).at[:feat, :].set(Wl1.T)
    wr1t = jnp.zeros((F_PAD, HID), f32).at[:feat, :].set(Wr1.T)
    RB = 512
    nrb = N2 // RB
    h1 = pl.pallas_call(
        _tc_h1_body,
        grid=(nrb,),
        in_specs=[pl.BlockSpec((RB, LANES), lambda j: (j, 0))
                  for _ in range(8)] + [
            pl.BlockSpec((RB, F_PAD), lambda j: (j, 0)),
            pl.BlockSpec((F_PAD, HID), lambda j: (0, 0)),
            pl.BlockSpec((F_PAD, HID), lambda j: (0, 0)),
            pl.BlockSpec((1, HID), lambda j: (0, 0)),
        ],
        out_specs=pl.BlockSpec((RB, HID), lambda j: (j, 0)),
        out_shape=jax.ShapeDtypeStruct((N2, HID), f32),
    )(agg1[0, 0], agg1[0, 1], agg1[0, 2], agg1[0, 3],
      agg1[1, 0], agg1[1, 1], agg1[1, 2], agg1[1, 3],
      xpad, wl1t, wr1t, bl1.reshape(1, HID))

    zeros2d = jnp.zeros((T2_PT, HID), f32)
    conv2 = pl.kernel(
        functools.partial(_sc_conv2, eblocks2=eblocks),
        out_type=jax.ShapeDtypeStruct((NC * T2, HID), f32),
        mesh=mesh,
        scratch_types=[
            pltpu.VMEM((EBLK,), jnp.int32),
            pltpu.VMEM((EBLK,), jnp.int32),
            pltpu.VMEM((SB,), jnp.int32),
            pltpu.VMEM((SB,), jnp.int32),
            pltpu.VMEM((SB, HID), f32),
            pltpu.VMEM_SHARED((T2, HID), f32),
        ],
    )
    agg2 = conv2(src, dst, h1, zeros2d)
    cnt2 = agg1[0, NCOL - 1, :NSEL, LANES - 1:] + agg1[1, NCOL - 1, :NSEL, LANES - 1:]

    wl2t = jnp.zeros((HID, F_PAD), f32).at[:, :feat].set(Wl2.T)
    wr2t = jnp.zeros((HID, F_PAD), f32).at[:, :feat].set(Wr2.T)
    bl2p = jnp.zeros((1, F_PAD), f32).at[0, :feat].set(bl2)
    w1t = jnp.zeros((F_PAD, HID), f32).at[:feat, :].set(W1.T)
    w3t = jnp.zeros((HID, 8), f32).at[:, 0].set(W3[0])
    b3p = jnp.zeros((1, 8), f32).at[0, 0].set(b3[0])
    rmask = (jnp.arange(NSEL) < action_range).astype(f32).reshape(NSEL, 1)

    out8 = pl.pallas_call(
        _tc_head_body,
        out_shape=jax.ShapeDtypeStruct((NSEL, 8), f32),
    )(agg2, cnt2, h1[:NSEL], rmask,
      wl2t, bl2p, wr2t, w1t, b1.reshape(1, HID),
      W2.T, b2.reshape(1, HID), w3t, b3p)

    return out8[:, 0].reshape(1, NSEL)


# speedup vs baseline: 1.0134x; 1.0134x over previous
import functools

import jax
import jax.numpy as jnp
from jax import lax
from jax.experimental import pallas as pl
from jax.experimental.pallas import tpu as pltpu
from jax.experimental.pallas import tpu_sc as plsc

NC = 2
NS = 16
LANES = 16

NSEL = 1000
F_PAD = 64
NCOL = F_PAD // LANES
HID = 128

N2 = 102400
ROWS_PT = N2 // NS
ZR = 640
EBLK = 2048
SB = 128
PIPE1 = 6
PIPE2 = 3
T2 = 1024
T2_PT = T2 // NS


def _sc_conv1(src_h, dst2d_h, xc0_h, xc1_h, xc2_h, xc3_h, zeros_h,
              agg_out, src1d, dst2d, rows_buf, gsem, ssem, sh_agg, *, eblocks):
    c = lax.axis_index("c")
    s = lax.axis_index("s")
    ept = eblocks * EBLK
    tbase = (c * NS + s) * ept
    xc = [xc0_h, xc1_h, xc2_h, xc3_h]

    for p in range(NCOL):
        def _z(z, _):
            pltpu.sync_copy(zeros_h,
                            sh_agg.at[pl.ds(s * ROWS_PT + z * ZR, ZR), :])
            return 0
        lax.fori_loop(0, ROWS_PT // ZR, _z, 0)
        plsc.subcore_barrier()

        def _blk(b, _):
            eb = tbase + b * EBLK
            rb = pl.multiple_of(eb // SB, LANES)
            pltpu.sync_copy(src_h.at[pl.ds(eb, EBLK)], src1d)
            pltpu.sync_copy(dst2d_h.at[pl.ds(rb, LANES), :], dst2d)
            NJ = EBLK // SB
            L = PIPE1
            gd, sd = {}, {}

            def fire_g(j):
                gd[j] = pltpu.async_copy(
                    xc[p].at[src1d.at[pl.ds(j * SB, SB)]],
                    rows_buf.at[j % (2 * L)], gsem)

            def fire_s(j):
                sd[j] = pltpu.async_copy(
                    rows_buf.at[j % (2 * L)],
                    sh_agg.at[dst2d.at[j]], ssem, add=True)

            for j in range(min(L, NJ)):
                fire_g(j)
            for j in range(NJ):
                gd[j].wait()
                fire_s(j)
                nj = j + L
                if nj < NJ:
                    pj = j - L
                    if pj >= 0:
                        sd.pop(pj).wait()
                    fire_g(nj)
            for j in sorted(sd):
                sd[j].wait()
            return 0
        lax.fori_loop(0, eblocks, _blk, 0)
        plsc.subcore_barrier()

        ob = pl.multiple_of((c * NCOL + p) * N2 + s * ROWS_PT, 8)
        pltpu.sync_copy(sh_agg.at[pl.ds(s * ROWS_PT, ROWS_PT), :],
                        agg_out.at[pl.ds(ob, ROWS_PT), :])
        plsc.subcore_barrier()


def _sc_conv2(src_h, dst_h, h1_h, zeros2d_h,
              agg_out,
              src1d, dst1d, cg_stage, cs_stage, rows_buf, gsem, ssem,
              sh_agg, *, eblocks2):
    c = lax.axis_index("c")
    s = lax.axis_index("s")
    ept = eblocks2 * EBLK
    tbase = (c * NS + s) * ept

    pltpu.sync_copy(zeros2d_h, sh_agg.at[pl.ds(s * T2_PT, T2_PT), :])
    plsc.subcore_barrier()

    def _blk(b, _):
        eb = tbase + b * EBLK
        pltpu.sync_copy(src_h.at[pl.ds(eb, EBLK)], src1d)
        pltpu.sync_copy(dst_h.at[pl.ds(eb, EBLK)], dst1d)
        NJ = EBLK // SB
        L = PIPE2
        gd, sd = {}, {}

        def fire_g(j):
            q = j % (2 * L)
            for i in range(SB // LANES):
                d = dst1d[pl.ds(j * SB + i * LANES, LANES)]
                sv = src1d[pl.ds(j * SB + i * LANES, LANES)]
                m = d < NSEL
                cg_stage[q, pl.ds(i * LANES, LANES)] = jnp.where(m, sv, 0)
                cs_stage[q, pl.ds(i * LANES, LANES)] = jnp.where(m, d, NSEL)
            gd[j] = pltpu.async_copy(h1_h.at[cg_stage.at[q]],
                                     rows_buf.at[q], gsem)

        def fire_s(j):
            q = j % (2 * L)
            sd[j] = pltpu.async_copy(rows_buf.at[q],
                                     sh_agg.at[cs_stage.at[q]], ssem, add=True)

        for j in range(min(L, NJ)):
            fire_g(j)
        for j in range(NJ):
            gd[j].wait()
            fire_s(j)
            nj = j + L
            if nj < NJ:
                pj = j - L
                if pj >= 0:
                    sd.pop(pj).wait()
                fire_g(nj)
        for j in sorted(sd):
            sd[j].wait()
        return 0
    lax.fori_loop(0, eblocks2, _blk, 0)
    plsc.subcore_barrier()

    ob = c * T2 + s * T2_PT
    pltpu.sync_copy(sh_agg.at[pl.ds(s * T2_PT, T2_PT), :],
                    agg_out.at[pl.ds(ob, T2_PT), :])


def _tc_h1_body(a0_ref, a1_ref, a2_ref, a3_ref, b0_ref, b1_ref, b2_ref,
                b3_ref, x_ref, wl_ref, wr_ref, bias_ref, out_ref):
    a = jnp.concatenate(
        [a0_ref[...] + b0_ref[...], a1_ref[...] + b1_ref[...],
         a2_ref[...] + b2_ref[...], a3_ref[...] + b3_ref[...]], axis=1)
    cnt = jnp.maximum(a[:, F_PAD - 1:F_PAD], 1.0)
    am = a / cnt
    h = (jnp.dot(am, wl_ref[...], preferred_element_type=jnp.float32)
         + jnp.dot(x_ref[...], wr_ref[...], preferred_element_type=jnp.float32)
         + bias_ref[...])
    out_ref[...] = jnp.maximum(h, 0.0)


def _tc_head_body(agg_ref, cnt_ref, h1t_ref, rmask_ref,
                  wl2_ref, bl2_ref, wr2_ref, w1_ref, b1_ref,
                  w2_ref, b2_ref, w3_ref, b3_ref, out_ref):
    a = agg_ref[0:NSEL, :] + agg_ref[T2:T2 + NSEL, :]
    cnt = jnp.maximum(cnt_ref[...], 1.0)
    am = a / cnt
    h2 = (jnp.dot(am, wl2_ref[...], preferred_element_type=jnp.float32)
          + bl2_ref[...]
          + jnp.dot(h1t_ref[...], wr2_ref[...], preferred_element_type=jnp.float32))
    h2 = h2 * rmask_ref[...]
    y = jnp.maximum(jnp.dot(h2, w1_ref[...], preferred_element_type=jnp.float32)
                    + b1_ref[...], 0.0)
    y = jnp.maximum(jnp.dot(y, w2_ref[...], preferred_element_type=jnp.float32)
                    + b2_ref[...], 0.0)
    out_ref[...] = (jnp.dot(y, w3_ref[...], preferred_element_type=jnp.float32)
                    + b3_ref[...])


def kernel(edge_feat, edge_index, action_range, Wl1, bl1, Wr1, Wl2, bl2, Wr2,
           W1, b1, W2, b2, W3, b3):
    n, feat = edge_feat.shape
    e = edge_index.shape[1]
    f32 = jnp.float32

    grp = NC * NS * EBLK
    epad = ((e + grp - 1) // grp) * grp
    src = jnp.concatenate([edge_index[0], jnp.zeros((epad - e,), jnp.int32)])
    dst = jnp.concatenate([edge_index[1],
                           jnp.full((epad - e,), n, jnp.int32)])
    dst2d = dst.reshape(epad // SB, SB)
    eblocks = epad // (NC * NS * EBLK)

    xpad = jnp.zeros((N2, F_PAD), f32).at[:n, :feat].set(edge_feat)
    xpad = xpad.at[:, F_PAD - 1].set(1.0)
    xcols = [xpad[:, p * LANES:(p + 1) * LANES] for p in range(NCOL)]

    mesh = plsc.VectorSubcoreMesh(core_axis_name="c", subcore_axis_name="s",
                                  num_cores=NC, num_subcores=NS)

    conv1 = pl.kernel(
        functools.partial(_sc_conv1, eblocks=eblocks),
        out_type=jax.ShapeDtypeStruct((2 * NCOL * N2, LANES), f32),
        mesh=mesh,
        compiler_params=pltpu.CompilerParams(use_tc_tiling_on_sc=False),
        scratch_types=[
            pltpu.VMEM((EBLK,), jnp.int32),
            pltpu.VMEM((LANES, SB), jnp.int32),
            pltpu.VMEM((2 * PIPE1, SB, LANES), f32),
            pltpu.SemaphoreType.DMA,
            pltpu.SemaphoreType.DMA,
            pltpu.VMEM_SHARED((N2, LANES), f32),
        ],
    )
    zeros1 = jnp.zeros((ZR, LANES), f32)
    agg1 = conv1(src, dst2d, xcols[0], xcols[1], xcols[2], xcols[3], zeros1)
    agg1 = agg1.reshape(2, NCOL, N2, LANES)

    wl1t = jnp.zeros((F_PAD, HID), f32).at[:feat, :].set(Wl1.T)
    wr1t = jnp.zeros((F_PAD, HID), f32).at[:feat, :].set(Wr1.T)
    RB = 512
    nrb = N2 // RB
    h1 = pl.pallas_call(
        _tc_h1_body,
        grid=(nrb,),
        in_specs=[pl.BlockSpec((RB, LANES), lambda j: (j, 0))
                  for _ in range(8)] + [
            pl.BlockSpec((RB, F_PAD), lambda j: (j, 0)),
            pl.BlockSpec((F_PAD, HID), lambda j: (0, 0)),
            pl.BlockSpec((F_PAD, HID), lambda j: (0, 0)),
            pl.BlockSpec((1, HID), lambda j: (0, 0)),
        ],
        out_specs=pl.BlockSpec((RB, HID), lambda j: (j, 0)),
        out_shape=jax.ShapeDtypeStruct((N2, HID), f32),
    )(agg1[0, 0], agg1[0, 1], agg1[0, 2], agg1[0, 3],
      agg1[1, 0], agg1[1, 1], agg1[1, 2], agg1[1, 3],
      xpad, wl1t, wr1t, bl1.reshape(1, HID))

    zeros2d = jnp.zeros((T2_PT, HID), f32)
    conv2 = pl.kernel(
        functools.partial(_sc_conv2, eblocks2=eblocks),
        out_type=jax.ShapeDtypeStruct((NC * T2, HID), f32),
        mesh=mesh,
        scratch_types=[
            pltpu.VMEM((EBLK,), jnp.int32),
            pltpu.VMEM((EBLK,), jnp.int32),
            pltpu.VMEM((2 * PIPE2, SB), jnp.int32),
            pltpu.VMEM((2 * PIPE2, SB), jnp.int32),
            pltpu.VMEM((2 * PIPE2, SB, HID), f32),
            pltpu.SemaphoreType.DMA,
            pltpu.SemaphoreType.DMA,
            pltpu.VMEM_SHARED((T2, HID), f32),
        ],
    )
    agg2 = conv2(src, dst, h1, zeros2d)
    cnt2 = agg1[0, NCOL - 1, :NSEL, LANES - 1:] + agg1[1, NCOL - 1, :NSEL, LANES - 1:]

    wl2t = jnp.zeros((HID, F_PAD), f32).at[:, :feat].set(Wl2.T)
    wr2t = jnp.zeros((HID, F_PAD), f32).at[:, :feat].set(Wr2.T)
    bl2p = jnp.zeros((1, F_PAD), f32).at[0, :feat].set(bl2)
    w1t = jnp.zeros((F_PAD, HID), f32).at[:feat, :].set(W1.T)
    w3t = jnp.zeros((HID, 8), f32).at[:, 0].set(W3[0])
    b3p = jnp.zeros((1, 8), f32).at[0, 0].set(b3[0])
    rmask = (jnp.arange(NSEL) < action_range).astype(f32).reshape(NSEL, 1)

    out8 = pl.pallas_call(
        _tc_head_body,
        out_shape=jax.ShapeDtypeStruct((NSEL, 8), f32),
    )(agg2, cnt2, h1[:NSEL], rmask,
      wl2t, bl2p, wr2t, w1t, b1.reshape(1, HID),
      W2.T, b2.reshape(1, HID), w3t, b3p)

    return out8[:, 0].reshape(1, NSEL)
